# Initial kernel scaffold; baseline (speedup 1.0000x reference)
import jax
import jax.numpy as jnp
from jax.experimental import pallas as pl


def kernel(x, ptr, W_sn_c, b_sn_c, W_sn_a, b_sn_a, Wc0, bc0, Wc1, bc1, Wc2, bc2, Wc3, bc3, Wa0, ba0, Wa1, ba1, Wa2, ba2, Wa3, ba3):
    raise NotImplementedError("write your pallas kernel here")


# speedup vs baseline: 2.7929x; 2.7929x over previous
import functools

import jax
import jax.numpy as jnp
from jax import lax
from jax.experimental import pallas as pl
from jax.experimental.pallas import tpu as pltpu

B = 16
MAX = 512
D = 128
NT = 10
H = 1024
N = 4096
TRI = MAX * (MAX + 1) // 2
OUT2 = MAX * (NT + 1)
NPAD = N + MAX + 128
TRIP = TRI + 256


def _elu(v):
    return jnp.where(v > 0, v, jnp.exp(jnp.minimum(v, 0.0)) - 1.0)


def _pad_body(ptr_ref, x_ref, wc_ref, wa_ref, bc_ref, ba_ref,
              outc_ref, outa_ref, sc_ref, sa_ref):
    dn = (((0,), (1,)), ((), ()))
    snc = lax.dot_general(wc_ref[...], x_ref[...], dn,
                          preferred_element_type=jnp.float32) + bc_ref[0, 0]
    sna = lax.dot_general(wa_ref[...], x_ref[...], dn,
                          preferred_element_type=jnp.float32) + ba_ref[0, 0]
    sc_ref[:, :N] = snc
    sc_ref[:, N:] = jnp.zeros((1, NPAD - N), jnp.float32)
    sa_ref[:, :N] = sna
    sa_ref[:, N:] = jnp.zeros((1, NPAD - N), jnp.float32)
    pos = lax.broadcasted_iota(jnp.int32, (1, MAX), 1)

    def body(b, carry):
        lo = ptr_ref[b]
        hi = ptr_ref[b + 1]
        m = pos < (hi - lo)
        off = lo % 128
        al = pl.multiple_of(lo - off, 128)
        vc = pltpu.roll(sc_ref[:, pl.ds(al, MAX + 128)],
                        (MAX + 128) - off, 1)[:, :MAX]
        va = pltpu.roll(sa_ref[:, pl.ds(al, MAX + 128)],
                        (MAX + 128) - off, 1)[:, :MAX]
        outc_ref[pl.ds(b, 1), :] = jnp.where(m, vc, 0.0)
        outa_ref[pl.ds(b, 1), :] = jnp.where(m, va, 0.0)
        return carry

    lax.fori_loop(0, B, body, 0)


def _pad_call(ptr, x, W_sn_c, W_sn_a, b_sn_c, b_sn_a):
    return pl.pallas_call(
        _pad_body,
        in_specs=[
            pl.BlockSpec(memory_space=pltpu.SMEM),
            pl.BlockSpec(memory_space=pltpu.MemorySpace.VMEM),
            pl.BlockSpec(memory_space=pltpu.MemorySpace.VMEM),
            pl.BlockSpec(memory_space=pltpu.MemorySpace.VMEM),
            pl.BlockSpec(memory_space=pltpu.MemorySpace.VMEM),
            pl.BlockSpec(memory_space=pltpu.MemorySpace.VMEM),
        ],
        out_shape=[
            jax.ShapeDtypeStruct((B, MAX), jnp.float32),
            jax.ShapeDtypeStruct((B, MAX), jnp.float32),
        ],
        scratch_shapes=[
            pltpu.VMEM((1, NPAD), jnp.float32),
            pltpu.VMEM((1, NPAD), jnp.float32),
        ],
    )(ptr, x, W_sn_c, W_sn_a,
      b_sn_c.reshape(1, 1), b_sn_a.reshape(1, 1))


def _mlp_body(pc_ref, pa_ref,
              wc0, bc0, wc1, bc1, wc2, bc2,
              wa0, ba0, wa1, ba1, wa2, ba2,
              h_ref, g_ref):
    h = _elu(jnp.dot(pc_ref[...], wc0[...],
                     preferred_element_type=jnp.float32) + bc0[...])
    h = _elu(jnp.dot(h, wc1[...],
                     preferred_element_type=jnp.float32) + bc1[...])
    h = _elu(jnp.dot(h, wc2[...],
                     preferred_element_type=jnp.float32) + bc2[...])
    h_ref[...] = h
    g = _elu(jnp.dot(pa_ref[...], wa0[...],
                     preferred_element_type=jnp.float32) + ba0[...])
    g = _elu(jnp.dot(g, wa1[...],
                     preferred_element_type=jnp.float32) + ba1[...])
    g = _elu(jnp.dot(g, wa2[...],
                     preferred_element_type=jnp.float32) + ba2[...])
    g_ref[...] = g


def _mlp_call(pad_c, pad_a, Wc0, bc0, Wc1, bc1, Wc2, bc2,
              Wa0, ba0, Wa1, ba1, Wa2, ba2):
    return pl.pallas_call(
        _mlp_body,
        out_shape=[
            jax.ShapeDtypeStruct((B, H), jnp.float32),
            jax.ShapeDtypeStruct((B, H), jnp.float32),
        ],
    )(pad_c, pad_a,
      Wc0, bc0.reshape(1, H), Wc1, bc1.reshape(1, H), Wc2, bc2.reshape(1, H),
      Wa0, ba0.reshape(1, H), Wa1, ba1.reshape(1, H), Wa2, ba2.reshape(1, H))


def _head_body(h_ref, w_ref, b_ref, o_ref):
    o_ref[...] = jnp.dot(h_ref[...], w_ref[...],
                         preferred_element_type=jnp.float32) + b_ref[...]


def _head_call(h, W, bvec, cols, cblk):
    nblk = cols // cblk
    return pl.pallas_call(
        _head_body,
        grid=(nblk,),
        in_specs=[
            pl.BlockSpec((B, H), lambda i: (0, 0)),
            pl.BlockSpec((H, cblk), lambda i: (0, i)),
            pl.BlockSpec((1, cblk), lambda i: (0, i)),
        ],
        out_specs=pl.BlockSpec((B, cblk), lambda i: (0, i)),
        out_shape=jax.ShapeDtypeStruct((B, cols), jnp.float32),
        compiler_params=pltpu.CompilerParams(
            dimension_semantics=("arbitrary",)),
    )(h, W, bvec.reshape(1, cols))


def _expand_body(d_ref, o_ref, u_ref, dp_ref):
    dp_ref[:, :TRI] = d_ref[0]

    def body(r, carry):
        base = r * MAX - (r * (r - 1)) // 2
        s = base - r
        off = s % 128
        al = pl.multiple_of(s - off, 128)
        sl = pltpu.roll(dp_ref[:, pl.ds(al, MAX + 128)],
                        (MAX + 128) - off, 1)[:, :MAX]
        c = lax.broadcasted_iota(jnp.int32, (1, MAX), 1)
        u_ref[pl.ds(r, 1), :] = jnp.where(c >= r, sl, 0.0)
        return carry

    lax.fori_loop(0, MAX, body, 0)
    u = u_ref[...]
    rr = lax.broadcasted_iota(jnp.int32, (MAX, MAX), 0)
    cc = lax.broadcasted_iota(jnp.int32, (MAX, MAX), 1)
    o_ref[...] = (u + jnp.where(rr > cc, u.T, 0.0))[None]


def _expand_call(x1_diag):
    return pl.pallas_call(
        _expand_body,
        grid=(B,),
        in_specs=[pl.BlockSpec((1, 1, TRI), lambda i: (i, 0, 0))],
        out_specs=pl.BlockSpec((1, MAX, MAX), lambda i: (i, 0, 0)),
        out_shape=jax.ShapeDtypeStruct((B, MAX, MAX), jnp.float32),
        scratch_shapes=[pltpu.VMEM((MAX, MAX), jnp.float32),
                        pltpu.VMEM((1, TRIP), jnp.float32)],
        compiler_params=pltpu.CompilerParams(
            dimension_semantics=("arbitrary",)),
    )(x1_diag.reshape(B, 1, TRI))


def kernel(x, ptr, W_sn_c, b_sn_c, W_sn_a, b_sn_a,
           Wc0, bc0, Wc1, bc1, Wc2, bc2, Wc3, bc3,
           Wa0, ba0, Wa1, ba1, Wa2, ba2, Wa3, ba3):
    pad_c, pad_a = _pad_call(ptr, x, W_sn_c, W_sn_a, b_sn_c, b_sn_a)
    h, g = _mlp_call(pad_c, pad_a, Wc0, bc0, Wc1, bc1, Wc2, bc2,
                     Wa0, ba0, Wa1, ba1, Wa2, ba2)
    x1_diag = _head_call(h, Wc3, bc3, TRI, 256)
    x2 = _head_call(g, Wa3, ba3, OUT2, 256)
    x1 = _expand_call(x1_diag)
    return (x1.reshape(B, MAX * MAX), x2)


# speedup vs baseline: 6.4040x; 2.2930x over previous
import functools

import jax
import jax.numpy as jnp
from jax import lax
from jax.experimental import pallas as pl
from jax.experimental.pallas import tpu as pltpu
from jax.experimental.pallas import tpu_sc as plsc

B = 16
MAX = 512
D = 128
NT = 10
H = 1024
N = 4096
TRI = MAX * (MAX + 1) // 2
OUT2 = MAX * (NT + 1)
NPAD = N + MAX + 128
TRIP = TRI + 256


def _elu(v):
    return jnp.where(v > 0, v, jnp.exp(jnp.minimum(v, 0.0)) - 1.0)


def _pad_body(ptr_ref, x_ref, wc_ref, wa_ref, bc_ref, ba_ref,
              outc_ref, outa_ref, sc_ref, sa_ref):
    dn = (((0,), (1,)), ((), ()))
    snc = lax.dot_general(wc_ref[...], x_ref[...], dn,
                          preferred_element_type=jnp.float32) + bc_ref[0, 0]
    sna = lax.dot_general(wa_ref[...], x_ref[...], dn,
                          preferred_element_type=jnp.float32) + ba_ref[0, 0]
    sc_ref[:, :N] = snc
    sc_ref[:, N:] = jnp.zeros((1, NPAD - N), jnp.float32)
    sa_ref[:, :N] = sna
    sa_ref[:, N:] = jnp.zeros((1, NPAD - N), jnp.float32)
    pos = lax.broadcasted_iota(jnp.int32, (1, MAX), 1)

    def body(b, carry):
        lo = ptr_ref[b]
        hi = ptr_ref[b + 1]
        m = pos < (hi - lo)
        off = lo % 128
        al = pl.multiple_of(lo - off, 128)
        vc = pltpu.roll(sc_ref[:, pl.ds(al, MAX + 128)],
                        (MAX + 128) - off, 1)[:, :MAX]
        va = pltpu.roll(sa_ref[:, pl.ds(al, MAX + 128)],
                        (MAX + 128) - off, 1)[:, :MAX]
        outc_ref[pl.ds(b, 1), :] = jnp.where(m, vc, 0.0)
        outa_ref[pl.ds(b, 1), :] = jnp.where(m, va, 0.0)
        return carry

    lax.fori_loop(0, B, body, 0)


def _pad_call(ptr, x, W_sn_c, W_sn_a, b_sn_c, b_sn_a):
    return pl.pallas_call(
        _pad_body,
        in_specs=[
            pl.BlockSpec(memory_space=pltpu.SMEM),
            pl.BlockSpec(memory_space=pltpu.MemorySpace.VMEM),
            pl.BlockSpec(memory_space=pltpu.MemorySpace.VMEM),
            pl.BlockSpec(memory_space=pltpu.MemorySpace.VMEM),
            pl.BlockSpec(memory_space=pltpu.MemorySpace.VMEM),
            pl.BlockSpec(memory_space=pltpu.MemorySpace.VMEM),
        ],
        out_shape=[
            jax.ShapeDtypeStruct((B, MAX), jnp.float32),
            jax.ShapeDtypeStruct((B, MAX), jnp.float32),
        ],
        scratch_shapes=[
            pltpu.VMEM((1, NPAD), jnp.float32),
            pltpu.VMEM((1, NPAD), jnp.float32),
        ],
    )(ptr, x, W_sn_c, W_sn_a,
      b_sn_c.reshape(1, 1), b_sn_a.reshape(1, 1))


def _mlp_body(pc_ref, pa_ref,
              wc0, bc0, wc1, bc1, wc2, bc2,
              wa0, ba0, wa1, ba1, wa2, ba2,
              h_ref, g_ref):
    h = _elu(jnp.dot(pc_ref[...], wc0[...],
                     preferred_element_type=jnp.float32) + bc0[...])
    h = _elu(jnp.dot(h, wc1[...],
                     preferred_element_type=jnp.float32) + bc1[...])
    h = _elu(jnp.dot(h, wc2[...],
                     preferred_element_type=jnp.float32) + bc2[...])
    h_ref[...] = h
    g = _elu(jnp.dot(pa_ref[...], wa0[...],
                     preferred_element_type=jnp.float32) + ba0[...])
    g = _elu(jnp.dot(g, wa1[...],
                     preferred_element_type=jnp.float32) + ba1[...])
    g = _elu(jnp.dot(g, wa2[...],
                     preferred_element_type=jnp.float32) + ba2[...])
    g_ref[...] = g


def _mlp_call(pad_c, pad_a, Wc0, bc0, Wc1, bc1, Wc2, bc2,
              Wa0, ba0, Wa1, ba1, Wa2, ba2):
    return pl.pallas_call(
        _mlp_body,
        out_shape=[
            jax.ShapeDtypeStruct((B, H), jnp.float32),
            jax.ShapeDtypeStruct((B, H), jnp.float32),
        ],
    )(pad_c, pad_a,
      Wc0, bc0.reshape(1, H), Wc1, bc1.reshape(1, H), Wc2, bc2.reshape(1, H),
      Wa0, ba0.reshape(1, H), Wa1, ba1.reshape(1, H), Wa2, ba2.reshape(1, H))


def _head_body(h_ref, w_ref, b_ref, o_ref):
    o_ref[...] = jnp.dot(h_ref[...], w_ref[...],
                         preferred_element_type=jnp.float32) + b_ref[...]


def _head_call(h, W, bvec, cols, cblk):
    nblk = cols // cblk
    return pl.pallas_call(
        _head_body,
        grid=(nblk,),
        in_specs=[
            pl.BlockSpec((B, H), lambda i: (0, 0)),
            pl.BlockSpec((H, cblk), lambda i: (0, i)),
            pl.BlockSpec((1, cblk), lambda i: (0, i)),
        ],
        out_specs=pl.BlockSpec((B, cblk), lambda i: (0, i)),
        out_shape=jax.ShapeDtypeStruct((B, cols), jnp.float32),
        compiler_params=pltpu.CompilerParams(
            dimension_semantics=("arbitrary",)),
    )(h, W, bvec.reshape(1, cols))


def _expand_body(d_ref, o_ref, u_ref, dp_ref):
    dp_ref[:, :TRI] = d_ref[0]

    def body(r, carry):
        base = r * MAX - (r * (r - 1)) // 2
        s = base - r
        off = s % 128
        al = pl.multiple_of(s - off, 128)
        sl = pltpu.roll(dp_ref[:, pl.ds(al, MAX + 128)],
                        (MAX + 128) - off, 1)[:, :MAX]
        c = lax.broadcasted_iota(jnp.int32, (1, MAX), 1)
        u_ref[pl.ds(r, 1), :] = jnp.where(c >= r, sl, 0.0)
        return carry

    lax.fori_loop(0, MAX, body, 0)
    u = u_ref[...]
    rr = lax.broadcasted_iota(jnp.int32, (MAX, MAX), 0)
    cc = lax.broadcasted_iota(jnp.int32, (MAX, MAX), 1)
    o_ref[...] = (u + jnp.where(rr > cc, u.T, 0.0))[None]


def _expand_call(x1_diag):
    return pl.pallas_call(
        _expand_body,
        grid=(B,),
        in_specs=[pl.BlockSpec((1, 1, TRI), lambda i: (i, 0, 0))],
        out_specs=pl.BlockSpec((1, MAX, MAX), lambda i: (i, 0, 0)),
        out_shape=jax.ShapeDtypeStruct((B, MAX, MAX), jnp.float32),
        scratch_shapes=[pltpu.VMEM((MAX, MAX), jnp.float32),
                        pltpu.VMEM((1, TRIP), jnp.float32)],
        compiler_params=pltpu.CompilerParams(
            dimension_semantics=("arbitrary",)),
    )(x1_diag.reshape(B, 1, TRI))


BLK = 128
NBLK = MAX // BLK
NW = 32

_COMBOS = [(R, C) for R in range(NBLK) for C in range(R, NBLK)]


def _tri_base(r):
    return r * MAX - (r * (r - 1)) // 2


def _stage_bounds(R, C):
    r0, c0 = R * BLK, C * BLK
    lo = _tri_base(r0) + c0 - r0
    hi = _tri_base(r0 + BLK - 1) + c0 - (r0 + BLK - 1) + BLK
    lo_al = lo & ~15
    hi_al = (hi + 15) & ~15
    return lo_al, hi_al - lo_al


_STAGE_MAX = max(_stage_bounds(R, C)[1] for R, C in _COMBOS)


def _expand_sc(x1_diag):
    mesh = plsc.VectorSubcoreMesh(core_axis_name="c", subcore_axis_name="s")

    @functools.partial(
        pl.kernel,
        mesh=mesh,
        out_type=jax.ShapeDtypeStruct((B, MAX, MAX), jnp.float32),
        compiler_params=pltpu.CompilerParams(needs_layout_passes=False),
        scratch_types=[
            pltpu.VMEM((_STAGE_MAX,), jnp.float32),
            pltpu.VMEM((BLK, BLK), jnp.float32),
            pltpu.VMEM((BLK, BLK), jnp.float32),
        ],
    )
    def k(d_hbm, o_hbm, stage, ubuf, tbuf):
        wid = lax.axis_index("s") * 2 + lax.axis_index("c")
        lane = lax.iota(jnp.int32, 16)
        for ci, (R, C) in enumerate(_COMBOS):
            half = ci % 2
            b = wid - 16 * half

            @pl.when((b >= 0) & (b < 16))
            def _task(R=R, C=C, b=b):
                r0, c0 = R * BLK, C * BLK
                lo_al, L = _stage_bounds(R, C)
                src0 = pl.multiple_of(b * TRI + lo_al, 8)
                pltpu.sync_copy(d_hbm.at[pl.ds(src0, L)],
                                stage.at[pl.ds(0, L)])

                def row(r_loc, carry):
                    r = r0 + r_loc
                    off = r * MAX - (r * (r - 1)) // 2 + c0 - r - lo_al
                    for kk in range(BLK // 16):
                        idx = off + kk * 16 + lane
                        v = plsc.load_gather(stage, [idx])
                        if R == C:
                            cc = c0 + kk * 16 + lane
                            v = jnp.where(cc >= r, v, 0.0)
                        ubuf[r_loc, pl.ds(kk * 16, 16)] = v
                        if R != C:
                            plsc.store_scatter(
                                tbuf, [kk * 16 + lane,
                                       jnp.broadcast_to(r_loc, (16,))], v)
                    return carry

                lax.fori_loop(0, BLK, row, 0)

                if R == C:
                    def row2(r_loc, carry):
                        r = r0 + r_loc
                        for kk in range(BLK // 16):
                            v = ubuf[r_loc, pl.ds(kk * 16, 16)]
                            cc = c0 + kk * 16 + lane
                            plsc.store_scatter(
                                ubuf, [kk * 16 + lane,
                                       jnp.broadcast_to(r_loc, (16,))],
                                v, mask=cc > r)
                        return carry

                    lax.fori_loop(0, BLK, row2, 0)
                    pltpu.sync_copy(
                        ubuf, o_hbm.at[b, pl.ds(r0, BLK), pl.ds(c0, BLK)])
                else:
                    pltpu.sync_copy(
                        ubuf, o_hbm.at[b, pl.ds(r0, BLK), pl.ds(c0, BLK)])
                    pltpu.sync_copy(
                        tbuf, o_hbm.at[b, pl.ds(c0, BLK), pl.ds(r0, BLK)])

    return k(x1_diag.reshape(B * TRI))


def kernel(x, ptr, W_sn_c, b_sn_c, W_sn_a, b_sn_a,
           Wc0, bc0, Wc1, bc1, Wc2, bc2, Wc3, bc3,
           Wa0, ba0, Wa1, ba1, Wa2, ba2, Wa3, ba3):
    pad_c, pad_a = _pad_call(ptr, x, W_sn_c, W_sn_a, b_sn_c, b_sn_a)
    h, g = _mlp_call(pad_c, pad_a, Wc0, bc0, Wc1, bc1, Wc2, bc2,
                     Wa0, ba0, Wa1, ba1, Wa2, ba2)
    x1_diag = _head_call(h, Wc3, bc3, TRI, 256)
    x2 = _head_call(g, Wa3, ba3, OUT2, 256)
    x1 = _expand_sc(x1_diag)
    return (x1.reshape(B, MAX * MAX), x2)
